# baseline (speedup 1.0000x reference)
import jax
import jax.numpy as jnp
from jax.experimental import pallas as pl
from jax.experimental.pallas import tpu as pltpu

B = 256
DV = 1024
DT = 768
DP = 384
E = 8
TOPK = 2
DFF = 2048
NC = 1000
TEMP = 0.1
SUB = 64

_SQRT_HALF = 0.7071067811865476


def _gelu(x):
    return x * 0.5 * (1.0 + jax.lax.erf(x * _SQRT_HALF))


def _router_kernel(text_ref, Wt_ref, Wm_ref, bm_ref, Wip_ref, bip_ref, noise_ref,
                   tproj_ref, rank_ref, counts_ref, il_ref, ent_ref):
    tf = jnp.dot(text_ref[...], Wt_ref[...], preferred_element_type=jnp.float32)
    logits = (jnp.dot(tf, Wm_ref[...], preferred_element_type=jnp.float32)
              + bm_ref[...]) / TEMP + noise_ref[...]
    mx = jnp.max(logits, axis=1, keepdims=True)
    ex = jnp.exp(logits - mx)
    scores = ex / jnp.sum(ex, axis=1, keepdims=True)
    iota = jax.lax.broadcasted_iota(jnp.int32, (B, E), 1)
    m1 = jnp.max(scores, axis=1, keepdims=True)
    i1 = jnp.min(jnp.where(scores == m1, iota, E), axis=1, keepdims=True)
    s2 = jnp.where(iota == i1, -jnp.inf, scores)
    m2 = jnp.max(s2, axis=1, keepdims=True)
    i2 = jnp.min(jnp.where(s2 == m2, iota, E), axis=1, keepdims=True)
    mask = ((iota == i1) | (iota == i2)).astype(jnp.float32)
    tri = (jax.lax.broadcasted_iota(jnp.int32, (B, B), 0)
           >= jax.lax.broadcasted_iota(jnp.int32, (B, B), 1)).astype(jnp.bfloat16)
    csum = jnp.dot(tri, mask.astype(jnp.bfloat16),
                   preferred_element_type=jnp.float32)
    rank_ref[...] = jnp.where(mask == 1.0, csum - 1.0, -1.0)
    counts_ref[...] = jnp.sum(mask, axis=0, keepdims=True).astype(jnp.int32)
    sum_scores = jnp.sum(scores, axis=0)
    mean_s = jnp.mean(sum_scores)
    var = jnp.sum((sum_scores - mean_s) ** 2) / (E - 1)
    il = var / (mean_s * mean_s)
    il_ref[0, 0] = jnp.where(il > 0.05, il, 0.0)
    ent_ref[0, 0] = jnp.mean(-jnp.sum(scores * jnp.log(scores + 1e-7), axis=1))
    tp = jnp.dot(tf, Wip_ref[...], preferred_element_type=jnp.float32) + bip_ref[...]
    tproj_ref[...] = _gelu(tp).astype(jnp.bfloat16)


def _expert_kernel(counts_ref, vis_ref, tproj_ref, rankT_ref,
                   W1v_ref, W1t_ref, b1_ref, W2_ref, b2_ref, out_ref, h_ref):
    e = pl.program_id(0)

    @pl.when(e == 0)
    def _init():
        out_ref[...] = jnp.zeros_like(out_ref)

    cnt = counts_ref[0, e]
    bf = jnp.bfloat16
    rr = rankT_ref[0]
    row_f = jax.lax.broadcasted_iota(jnp.int32, (B, 1), 0).astype(jnp.float32)
    disp = (rr == row_f).astype(bf)
    xv = jnp.dot(disp, vis_ref[...], preferred_element_type=jnp.float32).astype(bf)
    xt = jnp.dot(disp, tproj_ref[...], preferred_element_type=jnp.float32).astype(bf)
    pre = (jnp.dot(xv, W1v_ref[0].astype(bf), preferred_element_type=jnp.float32)
           + jnp.dot(xt, W1t_ref[0].astype(bf), preferred_element_type=jnp.float32)
           + b1_ref[0])
    h_ref[...] = pre

    def _act(j, carry):
        rows = pl.ds(j * SUB, SUB)
        h_ref[rows, :] = _gelu(h_ref[rows, :])
        return carry

    jax.lax.fori_loop(0, (cnt + SUB - 1) // SUB, _act, 0)

    part = jnp.dot(h_ref[...].astype(bf), W2_ref[0].astype(bf),
                   preferred_element_type=jnp.float32) + b2_ref[0]
    out_ref[...] += jax.lax.dot_general(
        disp, part.astype(bf), (((0,), (0,)), ((), ())),
        preferred_element_type=jnp.float32)


def kernel(vision_input, text_input, W_text, W_moe, b_moe, W_ip, b_ip,
           W1v, W1t, b1, W2, b2):
    noise = jax.random.normal(jax.random.key(42), (B, E), dtype=jnp.float32) / (E ** 2)

    tproj, rank, counts, il, ent = pl.pallas_call(
        _router_kernel,
        out_shape=[
            jax.ShapeDtypeStruct((B, DP), jnp.bfloat16),
            jax.ShapeDtypeStruct((B, E), jnp.float32),
            jax.ShapeDtypeStruct((1, E), jnp.int32),
            jax.ShapeDtypeStruct((1, 1), jnp.float32),
            jax.ShapeDtypeStruct((1, 1), jnp.float32),
        ],
        out_specs=[
            pl.BlockSpec((B, DP), lambda: (0, 0)),
            pl.BlockSpec((B, E), lambda: (0, 0)),
            pl.BlockSpec((1, E), lambda: (0, 0)),
            pl.BlockSpec(memory_space=pltpu.SMEM),
            pl.BlockSpec(memory_space=pltpu.SMEM),
        ],
    )(text_input, W_text, W_moe, b_moe.reshape(1, E), W_ip,
      b_ip.reshape(1, DP), noise)

    logits = pl.pallas_call(
        _expert_kernel,
        grid=(E,),
        in_specs=[
            pl.BlockSpec(memory_space=pltpu.SMEM),
            pl.BlockSpec((B, DV), lambda e: (0, 0)),
            pl.BlockSpec((B, DP), lambda e: (0, 0)),
            pl.BlockSpec((1, 1, B), lambda e: (e, 0, 0)),
            pl.BlockSpec((1, DV, DFF), lambda e: (e, 0, 0)),
            pl.BlockSpec((1, DP, DFF), lambda e: (e, 0, 0)),
            pl.BlockSpec((1, 1, DFF), lambda e: (e, 0, 0)),
            pl.BlockSpec((1, DFF, NC), lambda e: (e, 0, 0)),
            pl.BlockSpec((1, 1, NC), lambda e: (e, 0, 0)),
        ],
        out_specs=pl.BlockSpec((B, NC), lambda e: (0, 0)),
        out_shape=jax.ShapeDtypeStruct((B, NC), jnp.float32),
        scratch_shapes=[pltpu.VMEM((B, DFF), jnp.float32)],
    )(counts, vision_input.astype(jnp.bfloat16), tproj,
      rank.T.reshape(E, 1, B), W1v, W1t,
      b1.reshape(E, 1, DFF), W2, b2.reshape(E, 1, NC))

    return (logits, il.reshape(()), ent.reshape(()))


# speedup vs baseline: 1.0006x; 1.0006x over previous
import jax
import jax.numpy as jnp
from jax.experimental import pallas as pl
from jax.experimental.pallas import tpu as pltpu

B = 256
DV = 1024
DT = 768
DP = 384
E = 8
TOPK = 2
DFF = 2048
NC = 1000
TEMP = 0.1
SUB = 64

_SQRT_HALF = 0.7071067811865476


def _gelu(x):
    return x * 0.5 * (1.0 + jax.lax.erf(x * _SQRT_HALF))


def _router_kernel(text_ref, Wt_ref, Wm_ref, bm_ref, Wip_ref, bip_ref, noise_ref,
                   tproj_ref, rank_ref, il_ref, ent_ref):
    tf = jnp.dot(text_ref[...], Wt_ref[...], preferred_element_type=jnp.float32)
    logits = (jnp.dot(tf, Wm_ref[...], preferred_element_type=jnp.float32)
              + bm_ref[...]) / TEMP + noise_ref[...]
    mx = jnp.max(logits, axis=1, keepdims=True)
    ex = jnp.exp(logits - mx)
    scores = ex / jnp.sum(ex, axis=1, keepdims=True)
    iota = jax.lax.broadcasted_iota(jnp.int32, (B, E), 1)
    m1 = jnp.max(scores, axis=1, keepdims=True)
    i1 = jnp.min(jnp.where(scores == m1, iota, E), axis=1, keepdims=True)
    s2 = jnp.where(iota == i1, -jnp.inf, scores)
    m2 = jnp.max(s2, axis=1, keepdims=True)
    i2 = jnp.min(jnp.where(s2 == m2, iota, E), axis=1, keepdims=True)
    mask = ((iota == i1) | (iota == i2)).astype(jnp.float32)
    tri = (jax.lax.broadcasted_iota(jnp.int32, (B, B), 0)
           >= jax.lax.broadcasted_iota(jnp.int32, (B, B), 1)).astype(jnp.bfloat16)
    csum = jnp.dot(tri, mask.astype(jnp.bfloat16),
                   preferred_element_type=jnp.float32)
    rank = jnp.where(mask == 1.0, csum - 1.0, -1.0)
    eye = (jax.lax.broadcasted_iota(jnp.int32, (B, B), 0)
           == jax.lax.broadcasted_iota(jnp.int32, (B, B), 1)).astype(jnp.bfloat16)
    rank_ref[...] = jax.lax.dot_general(
        rank.astype(jnp.bfloat16), eye, (((0,), (0,)), ((), ())),
        preferred_element_type=jnp.float32)
    sum_scores = jnp.sum(scores, axis=0)
    mean_s = jnp.mean(sum_scores)
    var = jnp.sum((sum_scores - mean_s) ** 2) / (E - 1)
    il = var / (mean_s * mean_s)
    il_ref[0, 0] = jnp.where(il > 0.05, il, 0.0)
    ent_ref[0, 0] = jnp.mean(-jnp.sum(scores * jnp.log(scores + 1e-7), axis=1))
    tp = jnp.dot(tf, Wip_ref[...], preferred_element_type=jnp.float32) + bip_ref[...]
    tproj_ref[...] = _gelu(tp).astype(jnp.bfloat16)


def _expert_kernel(vis_ref, tproj_ref, rankT_ref,
                   W1v_ref, W1t_ref, b1_ref, W2_ref, b2_ref, out_ref, h_ref):
    e = pl.program_id(0)

    @pl.when(e == 0)
    def _init():
        out_ref[...] = jnp.zeros_like(out_ref)

    bf = jnp.bfloat16
    rr = rankT_ref[0]
    cnt = jnp.max(rr) + 1.0
    row_f = jax.lax.broadcasted_iota(jnp.int32, (B, 1), 0).astype(jnp.float32)
    disp = (rr == row_f).astype(bf)
    xv = jnp.dot(disp, vis_ref[...], preferred_element_type=jnp.float32).astype(bf)
    xt = jnp.dot(disp, tproj_ref[...], preferred_element_type=jnp.float32).astype(bf)
    pre = (jnp.dot(xv, W1v_ref[0].astype(bf), preferred_element_type=jnp.float32)
           + jnp.dot(xt, W1t_ref[0].astype(bf), preferred_element_type=jnp.float32)
           + b1_ref[0])
    h_ref[...] = pre

    def _act(j, carry):
        rows = pl.ds(j * SUB, SUB)
        h_ref[rows, :] = _gelu(h_ref[rows, :])
        return carry

    nblk = jnp.floor((cnt + (SUB - 1.0)) / SUB).astype(jnp.int32)
    jax.lax.fori_loop(0, nblk, _act, 0)

    part = jnp.dot(h_ref[...].astype(bf), W2_ref[0].astype(bf),
                   preferred_element_type=jnp.float32) + b2_ref[0]
    out_ref[...] += jax.lax.dot_general(
        disp, part.astype(bf), (((0,), (0,)), ((), ())),
        preferred_element_type=jnp.float32)


def kernel(vision_input, text_input, W_text, W_moe, b_moe, W_ip, b_ip,
           W1v, W1t, b1, W2, b2):
    noise = jax.random.normal(jax.random.key(42), (B, E), dtype=jnp.float32) / (E ** 2)

    tproj, rank_t, il, ent = pl.pallas_call(
        _router_kernel,
        out_shape=[
            jax.ShapeDtypeStruct((B, DP), jnp.bfloat16),
            jax.ShapeDtypeStruct((E, B), jnp.float32),
            jax.ShapeDtypeStruct((1, 1), jnp.float32),
            jax.ShapeDtypeStruct((1, 1), jnp.float32),
        ],
        out_specs=[
            pl.BlockSpec((B, DP), lambda: (0, 0)),
            pl.BlockSpec((E, B), lambda: (0, 0)),
            pl.BlockSpec(memory_space=pltpu.SMEM),
            pl.BlockSpec(memory_space=pltpu.SMEM),
        ],
    )(text_input, W_text, W_moe, b_moe.reshape(1, E), W_ip,
      b_ip.reshape(1, DP), noise)

    logits = pl.pallas_call(
        _expert_kernel,
        grid=(E,),
        in_specs=[
            pl.BlockSpec((B, DV), lambda e: (0, 0)),
            pl.BlockSpec((B, DP), lambda e: (0, 0)),
            pl.BlockSpec((1, 1, B), lambda e: (e, 0, 0)),
            pl.BlockSpec((1, DV, DFF), lambda e: (e, 0, 0)),
            pl.BlockSpec((1, DP, DFF), lambda e: (e, 0, 0)),
            pl.BlockSpec((1, 1, DFF), lambda e: (e, 0, 0)),
            pl.BlockSpec((1, DFF, NC), lambda e: (e, 0, 0)),
            pl.BlockSpec((1, 1, NC), lambda e: (e, 0, 0)),
        ],
        out_specs=pl.BlockSpec((B, NC), lambda e: (0, 0)),
        out_shape=jax.ShapeDtypeStruct((B, NC), jnp.float32),
        scratch_shapes=[pltpu.VMEM((B, DFF), jnp.float32)],
    )(vision_input.astype(jnp.bfloat16), tproj,
      rank_t.reshape(E, 1, B), W1v, W1t,
      b1.reshape(E, 1, DFF), W2, b2.reshape(E, 1, NC))

    return (logits, il.reshape(()), ent.reshape(()))


# speedup vs baseline: 1.0341x; 1.0335x over previous
import jax
import jax.numpy as jnp
from jax.experimental import pallas as pl
from jax.experimental.pallas import tpu as pltpu

B = 256
DV = 1024
DT = 768
DP = 384
E = 8
TOPK = 2
DFF = 2048
NC = 1000
TEMP = 0.1
SUB = 64

_SQRT_HALF = 0.7071067811865476


def _gelu(x):
    return x * 0.5 * (1.0 + jax.lax.erf(x * _SQRT_HALF))


def _moe_kernel(text_ref, Wt_ref, Wm_ref, bm_ref, Wip_ref, bip_ref, noise_ref,
                vis_ref, W1v_ref, W1t_ref, b1_ref, W2_ref, b2_ref,
                out_ref, il_ref, ent_ref,
                rankT_s, tproj_s, h_s):
    e = pl.program_id(0)
    bf = jnp.bfloat16

    @pl.when(e == 0)
    def _router():
        out_ref[...] = jnp.zeros_like(out_ref)
        tf = jnp.dot(text_ref[...], Wt_ref[...],
                     preferred_element_type=jnp.float32)
        logits = (jnp.dot(tf, Wm_ref[...], preferred_element_type=jnp.float32)
                  + bm_ref[...]) / TEMP + noise_ref[...]
        mx = jnp.max(logits, axis=1, keepdims=True)
        ex = jnp.exp(logits - mx)
        scores = ex / jnp.sum(ex, axis=1, keepdims=True)
        iota = jax.lax.broadcasted_iota(jnp.int32, (B, E), 1)
        m1 = jnp.max(scores, axis=1, keepdims=True)
        i1 = jnp.min(jnp.where(scores == m1, iota, E), axis=1, keepdims=True)
        s2 = jnp.where(iota == i1, -jnp.inf, scores)
        m2 = jnp.max(s2, axis=1, keepdims=True)
        i2 = jnp.min(jnp.where(s2 == m2, iota, E), axis=1, keepdims=True)
        mask = ((iota == i1) | (iota == i2)).astype(jnp.float32)
        tri = (jax.lax.broadcasted_iota(jnp.int32, (B, B), 0)
               >= jax.lax.broadcasted_iota(jnp.int32, (B, B), 1)).astype(bf)
        csum = jnp.dot(tri, mask.astype(bf), preferred_element_type=jnp.float32)
        rank = jnp.where(mask == 1.0, csum - 1.0, -1.0)
        eye = (jax.lax.broadcasted_iota(jnp.int32, (B, B), 0)
               == jax.lax.broadcasted_iota(jnp.int32, (B, B), 1)).astype(bf)
        rankT_s[...] = jax.lax.dot_general(
            rank.astype(bf), eye, (((0,), (0,)), ((), ())),
            preferred_element_type=jnp.float32)
        sum_scores = jnp.sum(scores, axis=0)
        mean_s = jnp.mean(sum_scores)
        var = jnp.sum((sum_scores - mean_s) ** 2) / (E - 1)
        il = var / (mean_s * mean_s)
        il_ref[0, 0] = jnp.where(il > 0.05, il, 0.0)
        ent_ref[0, 0] = jnp.mean(-jnp.sum(scores * jnp.log(scores + 1e-7), axis=1))
        tp = (jnp.dot(tf, Wip_ref[...], preferred_element_type=jnp.float32)
              + bip_ref[...])
        tproj_s[...] = _gelu(tp).astype(bf)

    rr = rankT_s[pl.ds(e, 1), :]
    cnt = jnp.max(rr) + 1.0
    row_f = jax.lax.broadcasted_iota(jnp.int32, (B, 1), 0).astype(jnp.float32)
    disp = (rr == row_f).astype(bf)
    xv = jnp.dot(disp, vis_ref[...].astype(bf),
                 preferred_element_type=jnp.float32).astype(bf)
    xt = jnp.dot(disp, tproj_s[...],
                 preferred_element_type=jnp.float32).astype(bf)
    h_s[...] = (jnp.dot(xv, W1v_ref[0].astype(bf),
                        preferred_element_type=jnp.float32)
                + jnp.dot(xt, W1t_ref[0].astype(bf),
                          preferred_element_type=jnp.float32)
                + b1_ref[0])

    def _act(j, carry):
        rows = pl.ds(j * SUB, SUB)
        h_s[rows, :] = _gelu(h_s[rows, :])
        return carry

    nblk = jnp.floor((cnt + (SUB - 1.0)) / SUB).astype(jnp.int32)
    jax.lax.fori_loop(0, nblk, _act, 0)

    part = jnp.dot(h_s[...].astype(bf), W2_ref[0].astype(bf),
                   preferred_element_type=jnp.float32) + b2_ref[0]
    out_ref[...] += jax.lax.dot_general(
        disp, part.astype(bf), (((0,), (0,)), ((), ())),
        preferred_element_type=jnp.float32)


def kernel(vision_input, text_input, W_text, W_moe, b_moe, W_ip, b_ip,
           W1v, W1t, b1, W2, b2):
    noise = jax.random.normal(jax.random.key(42), (B, E), dtype=jnp.float32) / (E ** 2)

    logits, il, ent = pl.pallas_call(
        _moe_kernel,
        grid=(E,),
        in_specs=[
            pl.BlockSpec((B, DT), lambda e: (0, 0)),
            pl.BlockSpec((DT, DT), lambda e: (0, 0)),
            pl.BlockSpec((DT, E), lambda e: (0, 0)),
            pl.BlockSpec((1, E), lambda e: (0, 0)),
            pl.BlockSpec((DT, DP), lambda e: (0, 0)),
            pl.BlockSpec((1, DP), lambda e: (0, 0)),
            pl.BlockSpec((B, E), lambda e: (0, 0)),
            pl.BlockSpec((B, DV), lambda e: (0, 0)),
            pl.BlockSpec((1, DV, DFF), lambda e: (e, 0, 0)),
            pl.BlockSpec((1, DP, DFF), lambda e: (e, 0, 0)),
            pl.BlockSpec((1, 1, DFF), lambda e: (e, 0, 0)),
            pl.BlockSpec((1, DFF, NC), lambda e: (e, 0, 0)),
            pl.BlockSpec((1, 1, NC), lambda e: (e, 0, 0)),
        ],
        out_specs=[
            pl.BlockSpec((B, NC), lambda e: (0, 0)),
            pl.BlockSpec(memory_space=pltpu.SMEM),
            pl.BlockSpec(memory_space=pltpu.SMEM),
        ],
        out_shape=[
            jax.ShapeDtypeStruct((B, NC), jnp.float32),
            jax.ShapeDtypeStruct((1, 1), jnp.float32),
            jax.ShapeDtypeStruct((1, 1), jnp.float32),
        ],
        scratch_shapes=[
            pltpu.VMEM((E, B), jnp.float32),
            pltpu.VMEM((B, DP), jnp.bfloat16),
            pltpu.VMEM((B, DFF), jnp.float32),
        ],
    )(text_input, W_text, W_moe, b_moe.reshape(1, E), W_ip,
      b_ip.reshape(1, DP), noise, vision_input, W1v, W1t,
      b1.reshape(E, 1, DFF), W2, b2.reshape(E, 1, NC))

    return (logits, il.reshape(()), ent.reshape(()))


# speedup vs baseline: 1.0373x; 1.0031x over previous
import jax
import jax.numpy as jnp
from jax.experimental import pallas as pl
from jax.experimental.pallas import tpu as pltpu

B = 256
DV = 1024
DT = 768
DP = 384
E = 8
TOPK = 2
DFF = 2048
NC = 1000
TEMP = 0.1
SUB = 64

_SQRT_HALF = 0.7071067811865476


def _gelu(x):
    return x * 0.5 * (1.0 + jax.lax.erf(x * _SQRT_HALF))


def _moe_kernel(text_ref, Wt_ref, Wm_ref, bm_ref, Wip_ref, bip_ref, noise_ref,
                vis_ref, W1v_ref, W1t_ref, b1_ref, W2_ref, b2_ref,
                out_ref, il_ref, ent_ref,
                rankT_s, tproj_s):
    e = pl.program_id(0)
    bf = jnp.bfloat16

    @pl.when(e == 0)
    def _router():
        out_ref[...] = jnp.zeros_like(out_ref)
        tf = jnp.dot(text_ref[...], Wt_ref[...],
                     preferred_element_type=jnp.float32)
        logits = (jnp.dot(tf, Wm_ref[...], preferred_element_type=jnp.float32)
                  + bm_ref[...]) / TEMP + noise_ref[...]
        mx = jnp.max(logits, axis=1, keepdims=True)
        ex = jnp.exp(logits - mx)
        scores = ex / jnp.sum(ex, axis=1, keepdims=True)
        iota = jax.lax.broadcasted_iota(jnp.int32, (B, E), 1)
        m1 = jnp.max(scores, axis=1, keepdims=True)
        i1 = jnp.min(jnp.where(scores == m1, iota, E), axis=1, keepdims=True)
        s2 = jnp.where(iota == i1, -jnp.inf, scores)
        m2 = jnp.max(s2, axis=1, keepdims=True)
        i2 = jnp.min(jnp.where(s2 == m2, iota, E), axis=1, keepdims=True)
        mask = ((iota == i1) | (iota == i2)).astype(jnp.float32)
        tri = (jax.lax.broadcasted_iota(jnp.int32, (B, B), 0)
               >= jax.lax.broadcasted_iota(jnp.int32, (B, B), 1)).astype(bf)
        csum = jnp.dot(tri, mask.astype(bf), preferred_element_type=jnp.float32)
        rank = jnp.where(mask == 1.0, csum - 1.0, -1.0)
        eye = (jax.lax.broadcasted_iota(jnp.int32, (B, B), 0)
               == jax.lax.broadcasted_iota(jnp.int32, (B, B), 1)).astype(bf)
        rankT_s[...] = jax.lax.dot_general(
            rank.astype(bf), eye, (((0,), (0,)), ((), ())),
            preferred_element_type=jnp.float32)
        sum_scores = jnp.sum(scores, axis=0)
        mean_s = jnp.mean(sum_scores)
        var = jnp.sum((sum_scores - mean_s) ** 2) / (E - 1)
        il = var / (mean_s * mean_s)
        il_ref[0, 0] = jnp.where(il > 0.05, il, 0.0)
        ent_ref[0, 0] = jnp.mean(-jnp.sum(scores * jnp.log(scores + 1e-7), axis=1))
        tp = (jnp.dot(tf, Wip_ref[...], preferred_element_type=jnp.float32)
              + bip_ref[...])
        tproj_s[...] = _gelu(tp).astype(bf)

    rr = rankT_s[pl.ds(e, 1), :]
    row_f = jax.lax.broadcasted_iota(jnp.int32, (B, 1), 0).astype(jnp.float32)
    disp = (rr == row_f).astype(bf)
    xv = jnp.dot(disp, vis_ref[...].astype(bf),
                 preferred_element_type=jnp.float32).astype(bf)
    xt = jnp.dot(disp, tproj_s[...],
                 preferred_element_type=jnp.float32).astype(bf)
    h = _gelu(jnp.dot(xv, W1v_ref[0].astype(bf),
                      preferred_element_type=jnp.float32)
              + jnp.dot(xt, W1t_ref[0].astype(bf),
                        preferred_element_type=jnp.float32)
              + b1_ref[0])
    part = jnp.dot(h.astype(bf), W2_ref[0].astype(bf),
                   preferred_element_type=jnp.float32) + b2_ref[0]
    out_ref[...] += jax.lax.dot_general(
        disp, part.astype(bf), (((0,), (0,)), ((), ())),
        preferred_element_type=jnp.float32)


def kernel(vision_input, text_input, W_text, W_moe, b_moe, W_ip, b_ip,
           W1v, W1t, b1, W2, b2):
    noise = jax.random.normal(jax.random.key(42), (B, E), dtype=jnp.float32) / (E ** 2)

    logits, il, ent = pl.pallas_call(
        _moe_kernel,
        grid=(E,),
        in_specs=[
            pl.BlockSpec((B, DT), lambda e: (0, 0)),
            pl.BlockSpec((DT, DT), lambda e: (0, 0)),
            pl.BlockSpec((DT, E), lambda e: (0, 0)),
            pl.BlockSpec((1, E), lambda e: (0, 0)),
            pl.BlockSpec((DT, DP), lambda e: (0, 0)),
            pl.BlockSpec((1, DP), lambda e: (0, 0)),
            pl.BlockSpec((B, E), lambda e: (0, 0)),
            pl.BlockSpec((B, DV), lambda e: (0, 0)),
            pl.BlockSpec((1, DV, DFF), lambda e: (e, 0, 0)),
            pl.BlockSpec((1, DP, DFF), lambda e: (e, 0, 0)),
            pl.BlockSpec((1, 1, DFF), lambda e: (e, 0, 0)),
            pl.BlockSpec((1, DFF, NC), lambda e: (e, 0, 0)),
            pl.BlockSpec((1, 1, NC), lambda e: (e, 0, 0)),
        ],
        out_specs=[
            pl.BlockSpec((B, NC), lambda e: (0, 0)),
            pl.BlockSpec(memory_space=pltpu.SMEM),
            pl.BlockSpec(memory_space=pltpu.SMEM),
        ],
        out_shape=[
            jax.ShapeDtypeStruct((B, NC), jnp.float32),
            jax.ShapeDtypeStruct((1, 1), jnp.float32),
            jax.ShapeDtypeStruct((1, 1), jnp.float32),
        ],
        scratch_shapes=[
            pltpu.VMEM((E, B), jnp.float32),
            pltpu.VMEM((B, DP), jnp.bfloat16),
        ],
    )(text_input, W_text, W_moe, b_moe.reshape(1, E), W_ip,
      b_ip.reshape(1, DP), noise, vision_input, W1v, W1t,
      b1.reshape(E, 1, DFF), W2, b2.reshape(E, 1, NC))

    return (logits, il.reshape(()), ent.reshape(()))
